# Initial kernel scaffold; baseline (speedup 1.0000x reference)
import jax
import jax.numpy as jnp
from jax.experimental import pallas as pl


def kernel(pos, emb, Rw0, Rb0, Rw1, Rb1, Rw2, Rb2, Rw3, Rb3, S0, S1, S2, S3, M1w, M1b, M2w, M2b, node_feats, edge_index):
    raise NotImplementedError("write your pallas kernel here")


# speedup vs baseline: 1.0118x; 1.0118x over previous
import functools

import jax
import jax.numpy as jnp
from jax.experimental import pallas as pl

NRBF = 8
C = 32
_CENTERS = [4.0 * k / (NRBF - 1) for k in range(NRBF)]


def _gather(table, idx):
    return table[idx]


def _segsum(data, idx, n):
    return jax.ops.segment_sum(data, idx, num_segments=n)


def _silu(x):
    return x * jax.nn.sigmoid(x)


def _dsilu(x):
    s = jax.nn.sigmoid(x)
    return s * (1.0 + x * (1.0 - s))


def _rbf(r):
    centers = jnp.asarray(_CENTERS, dtype=jnp.float32)
    return jnp.exp(-2.0 * (r[:, None] - centers[None, :]) ** 2)


def _sph2(u):
    x, y, z = u[:, 0], u[:, 1], u[:, 2]
    return jnp.stack([x * y, y * z, 3.0 * z * z - 1.0, x * z, x * x - y * y], axis=-1)


def manual_fwd_bwd(pos, emb, Rw0, Rb0, Rw1, Rb1, Rw2, Rb2, Rw3, Rb3,
                   S0, S1, S2, S3, M1w, M1b, M2w, M2b, node_feats, edge_index):
    N = pos.shape[0]
    src = edge_index[0]
    dst = edge_index[1]

    ps = _gather(pos, src)
    pd = _gather(pos, dst)
    vec = pd - ps
    r = jnp.sqrt(jnp.sum(vec * vec, axis=-1) + 1e-9)
    u = vec / r[:, None]
    B = _rbf(r)
    Y1 = u
    Y2 = _sph2(u)

    def rad(Rw, Rb, p):
        pre = B @ Rw[p] + Rb[p]
        return _silu(pre), pre

    h00 = _gather(emb, node_feats)
    h0s0 = _gather(h00, src)
    A0, preA0 = rad(Rw0, Rb0, 0)
    A1, preA1 = rad(Rw0, Rb0, 1)
    A2, preA2 = rad(Rw0, Rb0, 2)
    o0 = _segsum(A0 * h0s0, dst, N) + h00 @ S0[0]
    o1 = _segsum((A1 * h0s0)[:, :, None] * Y1[:, None, :], dst, N)
    o2 = _segsum((A2 * h0s0)[:, :, None] * Y2[:, None, :], dst, N)

    def layer_fwd(h0, h1, h2, Rw, Rb, S):
        h0s = _gather(h0, src)
        h1s = _gather(h1, src)
        h2s = _gather(h2, src)
        R = [rad(Rw, Rb, p)[0] for p in range(5)]
        inv1 = jnp.sum(h1s * Y1[:, None, :], axis=-1)
        inv2 = jnp.sum(h2s * Y2[:, None, :], axis=-1)
        m0 = R[0] * h0s + R[1] * inv1 + R[2] * inv2
        n0 = _segsum(m0, dst, N) + h0 @ S[0]
        n1 = _segsum((R[3] * h0s)[:, :, None] * Y1[:, None, :], dst, N) \
            + jnp.einsum('ncm,cd->ndm', h1, S[1])
        n2 = _segsum((R[4] * h0s)[:, :, None] * Y2[:, None, :], dst, N) \
            + jnp.einsum('ncm,cd->ndm', h2, S[2])
        return n0, n1, n2

    hA = (o0, o1, o2)
    hB = layer_fwd(*hA, Rw1, Rb1, S1)
    hC = layer_fwd(*hB, Rw2, Rb2, S2)

    h0s = _gather(hC[0], src)
    h1s = _gather(hC[1], src)
    h2s = _gather(hC[2], src)
    R3 = [rad(Rw3, Rb3, p) for p in range(3)]
    inv1 = jnp.sum(h1s * Y1[:, None, :], axis=-1)
    inv2 = jnp.sum(h2s * Y2[:, None, :], axis=-1)
    m0 = R3[0][0] * h0s + R3[1][0] * inv1 + R3[2][0] * inv2
    f0 = _segsum(m0, dst, N) + hC[0] @ S3[0]

    z = f0 @ M1w + M1b
    a = _silu(z)
    e = a @ M2w + M2b
    total = jnp.sum(e)

    g_a = jnp.broadcast_to(M2w[:, 0][None, :], a.shape)
    g_z = g_a * _dsilu(z)
    g_f0 = g_z @ M1w.T

    gB = jnp.zeros_like(B)
    gY1 = jnp.zeros_like(Y1)
    gY2 = jnp.zeros_like(Y2)

    gm0 = _gather(g_f0, dst)
    g_h0 = g_f0 @ S3[0].T
    g_h0s = gm0 * R3[0][0]
    g_inv1 = gm0 * R3[1][0]
    g_inv2 = gm0 * R3[2][0]
    g_h1s = g_inv1[:, :, None] * Y1[:, None, :]
    g_h2s = g_inv2[:, :, None] * Y2[:, None, :]
    gY1 = gY1 + jnp.sum(g_inv1[:, :, None] * h1s, axis=1)
    gY2 = gY2 + jnp.sum(g_inv2[:, :, None] * h2s, axis=1)
    for p, gR in ((0, gm0 * h0s), (1, gm0 * inv1), (2, gm0 * inv2)):
        gB = gB + (gR * _dsilu(R3[p][1])) @ Rw3[p].T
    g_h0 = g_h0 + _segsum(g_h0s, src, N)
    g_h1 = _segsum(g_h1s, src, N)
    g_h2 = _segsum(g_h2s, src, N)

    def layer_bwd(g_n0, g_n1, g_n2, h, Rw, Rb, S, gB, gY1, gY2):
        h0s = _gather(h[0], src)
        h1s = _gather(h[1], src)
        h2s = _gather(h[2], src)
        R = [rad(Rw, Rb, p) for p in range(5)]
        inv1 = jnp.sum(h1s * Y1[:, None, :], axis=-1)
        inv2 = jnp.sum(h2s * Y2[:, None, :], axis=-1)
        gm0 = _gather(g_n0, dst)
        gmsg1 = _gather(g_n1, dst)
        gmsg2 = _gather(g_n2, dst)
        q1 = jnp.sum(gmsg1 * Y1[:, None, :], axis=-1)
        q2 = jnp.sum(gmsg2 * Y2[:, None, :], axis=-1)
        g_h0s = gm0 * R[0][0] + q1 * R[3][0] + q2 * R[4][0]
        g_inv1 = gm0 * R[1][0]
        g_inv2 = gm0 * R[2][0]
        g_h1s = g_inv1[:, :, None] * Y1[:, None, :]
        g_h2s = g_inv2[:, :, None] * Y2[:, None, :]
        gY1 = gY1 + jnp.sum(g_inv1[:, :, None] * h1s, axis=1) \
            + jnp.sum(gmsg1 * (R[3][0] * h0s)[:, :, None], axis=1)
        gY2 = gY2 + jnp.sum(g_inv2[:, :, None] * h2s, axis=1) \
            + jnp.sum(gmsg2 * (R[4][0] * h0s)[:, :, None], axis=1)
        for p, gR in ((0, gm0 * h0s), (1, gm0 * inv1), (2, gm0 * inv2),
                      (3, q1 * h0s), (4, q2 * h0s)):
            gB = gB + (gR * _dsilu(R[p][1])) @ Rw[p].T
        g_h0p = _segsum(g_h0s, src, N) + g_n0 @ S[0].T
        g_h1p = _segsum(g_h1s, src, N) + jnp.einsum('ndm,cd->ncm', g_n1, S[1])
        g_h2p = _segsum(g_h2s, src, N) + jnp.einsum('ndm,cd->ncm', g_n2, S[2])
        return g_h0p, g_h1p, g_h2p, gB, gY1, gY2

    g_h0, g_h1, g_h2, gB, gY1, gY2 = layer_bwd(
        g_h0, g_h1, g_h2, hB, Rw2, Rb2, S2, gB, gY1, gY2)
    g_h0, g_h1, g_h2, gB, gY1, gY2 = layer_bwd(
        g_h0, g_h1, g_h2, hA, Rw1, Rb1, S1, gB, gY1, gY2)

    gm = _gather(g_h0, dst)
    gmsg1 = _gather(g_h1, dst)
    gmsg2 = _gather(g_h2, dst)
    q1 = jnp.sum(gmsg1 * Y1[:, None, :], axis=-1)
    q2 = jnp.sum(gmsg2 * Y2[:, None, :], axis=-1)
    gY1 = gY1 + jnp.sum(gmsg1 * (A1 * h0s0)[:, :, None], axis=1)
    gY2 = gY2 + jnp.sum(gmsg2 * (A2 * h0s0)[:, :, None], axis=1)
    for pre, Rw_p, gR in ((preA0, Rw0[0], gm * h0s0), (preA1, Rw0[1], q1 * h0s0),
                          (preA2, Rw0[2], q2 * h0s0)):
        gB = gB + (gR * _dsilu(pre)) @ Rw_p.T

    x, y, z_ = u[:, 0], u[:, 1], u[:, 2]
    gu = jnp.stack([
        gY1[:, 0] + gY2[:, 0] * y + gY2[:, 3] * z_ + 2.0 * x * gY2[:, 4],
        gY1[:, 1] + gY2[:, 0] * x + gY2[:, 1] * z_ - 2.0 * y * gY2[:, 4],
        gY1[:, 2] + gY2[:, 1] * y + 6.0 * z_ * gY2[:, 2] + gY2[:, 3] * x,
    ], axis=-1)
    centers = jnp.asarray(_CENTERS, dtype=jnp.float32)
    gr = jnp.sum(gB * (-4.0 * (r[:, None] - centers[None, :]) * B), axis=-1)
    gvec = gr[:, None] * u + (gu - u * jnp.sum(u * gu, axis=-1, keepdims=True)) / r[:, None]
    g_pos = _segsum(gvec, dst, N) - _segsum(gvec, src, N)
    forces = -g_pos
    return jnp.reshape(total, (1, 1)), forces


def kernel(pos, emb, Rw0, Rb0, Rw1, Rb1, Rw2, Rb2, Rw3, Rb3,
           S0, S1, S2, S3, M1w, M1b, M2w, M2b, node_feats, edge_index):
    return manual_fwd_bwd(pos, emb, Rw0, Rb0, Rw1, Rb1, Rw2, Rb2, Rw3, Rb3,
                          S0, S1, S2, S3, M1w, M1b, M2w, M2b, node_feats, edge_index)


# speedup vs baseline: 12.1888x; 12.0470x over previous
import functools

import jax
import jax.numpy as jnp
from jax import lax
from jax.experimental import pallas as pl
from jax.experimental.pallas import tpu as pltpu
from jax.experimental.pallas import tpu_sc as plsc

NRBF = 8
C = 32
_CENTERS = [4.0 * k / (NRBF - 1) for k in range(NRBF)]

_MESH = plsc.VectorSubcoreMesh(core_axis_name="c", subcore_axis_name="s")
_NW = 32
_CH = 128


@functools.lru_cache(maxsize=None)
def _make_sc_gather(V, D, M):
    assert M % _CH == 0 and (D * 4) % 64 == 0
    chunks = M // _CH
    cpw = -(-chunks // _NW)

    @functools.partial(
        pl.kernel, mesh=_MESH,
        compiler_params=pltpu.CompilerParams(use_tc_tiling_on_sc=False),
        out_type=jax.ShapeDtypeStruct((M, D), jnp.float32),
        scratch_types=[
            pltpu.VMEM((_CH,), jnp.int32),
            pltpu.VMEM((_CH, D), jnp.float32),
            pltpu.SemaphoreType.DMA,
        ],
    )
    def gather_k(table_hbm, idx_hbm, out_hbm, idx_v, rows_v, sem):
        w = lax.axis_index("s") * 2 + lax.axis_index("c")

        def body(i, carry):
            cid = w * cpw + i

            @pl.when(cid < chunks)
            def _():
                base = cid * _CH
                pltpu.sync_copy(idx_hbm.at[pl.ds(base, _CH)], idx_v)
                pltpu.async_copy(table_hbm.at[idx_v], rows_v, sem).wait()
                pltpu.sync_copy(rows_v, out_hbm.at[pl.ds(base, _CH)])
            return carry

        lax.fori_loop(0, cpw, body, 0)

    return gather_k


@functools.lru_cache(maxsize=None)
def _make_sc_segsum(P, M, N):
    assert M % (2 * _CH) == 0 and N % 16 == 0
    chunks = M // _CH
    per_core = chunks // 2
    cpt = -(-per_core // 16)
    NR = N // 16
    full, rem = NR // _CH, NR % _CH

    @functools.partial(
        pl.kernel, mesh=_MESH,
        compiler_params=pltpu.CompilerParams(use_tc_tiling_on_sc=False),
        out_type=jax.ShapeDtypeStruct((2, P, N, 32), jnp.float32),
        scratch_types=[
            pltpu.VMEM((1, _CH), jnp.int32),
            pltpu.VMEM((_CH, 32), jnp.float32),
            pltpu.VMEM((_CH, 32), jnp.float32),
            pltpu.VMEM_SHARED((N, 32), jnp.float32),
            pltpu.SemaphoreType.DMA,
        ],
    )
    def segsum_k(data_hbm, idx_hbm, out_hbm, idxb, datb, zb, acc, sem):
        c = lax.axis_index("c")
        s = lax.axis_index("s")

        def zrow(i, carry):
            zb[i // 2, pl.ds((i % 2) * 16, 16)] = jnp.zeros((16,), jnp.float32)
            return carry

        lax.fori_loop(0, 2 * _CH, zrow, 0)

        row0 = s * NR

        for plane in range(P):
            def zloop(i, carry):
                pltpu.sync_copy(zb, acc.at[pl.ds(row0 + i * _CH, _CH)])
                return carry

            lax.fori_loop(0, full, zloop, 0)
            if rem:
                pltpu.sync_copy(zb.at[pl.ds(0, rem)],
                                acc.at[pl.ds(row0 + full * _CH, rem)])
            plsc.subcore_barrier()

            def sloop(i, carry):
                cid_local = s * cpt + i

                @pl.when(cid_local < per_core)
                def _():
                    cid = c * per_core + cid_local
                    base = cid * _CH
                    pltpu.sync_copy(idx_hbm.at[pl.ds(cid, 1)], idxb)
                    pltpu.sync_copy(data_hbm.at[plane, pl.ds(base, _CH)], datb)
                    pltpu.sync_copy(datb, acc.at[idxb.at[0]], add=True)
                return carry

            lax.fori_loop(0, cpt, sloop, 0)
            plsc.subcore_barrier()

            def dloop(i, carry):
                pltpu.sync_copy(acc.at[pl.ds(row0 + i * _CH, _CH)],
                                out_hbm.at[c, plane, pl.ds(row0 + i * _CH, _CH)])
                return carry

            lax.fori_loop(0, full, dloop, 0)
            if rem:
                pltpu.sync_copy(acc.at[pl.ds(row0 + full * _CH, rem)],
                                out_hbm.at[c, plane, pl.ds(row0 + full * _CH, rem)])
            plsc.subcore_barrier()

    return segsum_k


def _pad_rows(a, mult):
    n = a.shape[0]
    pad = (-n) % mult
    if pad:
        a = jnp.concatenate([a, jnp.zeros((pad,) + a.shape[1:], a.dtype)], axis=0)
    return a, n


def _gather(table, idx):
    shp = table.shape
    t2 = table.reshape(shp[0], -1)
    D = t2.shape[1]
    if D % 16 != 0:
        t2 = jnp.concatenate(
            [t2, jnp.zeros((shp[0], (-D) % 16), t2.dtype)], axis=1)
    idx_p, m = _pad_rows(idx, _CH)
    out = _make_sc_gather(shp[0], t2.shape[1], idx_p.shape[0])(t2, idx_p)
    return out[:m, :D].reshape((m,) + shp[1:])


def _segsum(data, idx, n):
    shp = data.shape
    if data.ndim == 2:
        planes = data[None]
    else:
        planes = jnp.moveaxis(data, -1, 0)
    P, W = planes.shape[0], planes.shape[-1]
    if W != 32:
        planes = jnp.concatenate(
            [planes, jnp.zeros(planes.shape[:-1] + ((-W) % 32,), planes.dtype)],
            axis=-1)
    out = _make_sc_segsum(P, shp[0], n)(planes, _seg_idx2d(idx))
    tot = (out[0] + out[1])[..., :W]
    if data.ndim == 2:
        return tot[0]
    return jnp.moveaxis(tot, 0, -1)


def _seg_idx2d(idx):
    chunks = idx.shape[0] // _CH
    per_core = chunks // 2
    cpt = -(-per_core // 16)
    idx2 = idx.reshape(chunks, _CH)
    pad = 2 * 16 * cpt - chunks
    if pad:
        idx2 = jnp.concatenate([idx2, jnp.zeros((pad, _CH), idx.dtype)], axis=0)
    return idx2


def _silu(x):
    return x * jax.nn.sigmoid(x)


def _dsilu(x):
    s = jax.nn.sigmoid(x)
    return s * (1.0 + x * (1.0 - s))


def _rbf(r):
    centers = jnp.asarray(_CENTERS, dtype=jnp.float32)
    return jnp.exp(-2.0 * (r[:, None] - centers[None, :]) ** 2)


def _sph2(u):
    x, y, z = u[:, 0], u[:, 1], u[:, 2]
    return jnp.stack([x * y, y * z, 3.0 * z * z - 1.0, x * z, x * x - y * y], axis=-1)


def manual_fwd_bwd(pos, emb, Rw0, Rb0, Rw1, Rb1, Rw2, Rb2, Rw3, Rb3,
                   S0, S1, S2, S3, M1w, M1b, M2w, M2b, node_feats, edge_index):
    N = pos.shape[0]
    src = edge_index[0]
    dst = edge_index[1]

    ps = _gather(pos, src)
    pd = _gather(pos, dst)
    vec = pd - ps
    r = jnp.sqrt(jnp.sum(vec * vec, axis=-1) + 1e-9)
    u = vec / r[:, None]
    B = _rbf(r)
    Y1 = u
    Y2 = _sph2(u)

    def rad(Rw, Rb, p):
        pre = B @ Rw[p] + Rb[p]
        return _silu(pre), pre

    h00 = _gather(emb, node_feats)
    h0s0 = _gather(h00, src)
    A0, preA0 = rad(Rw0, Rb0, 0)
    A1, preA1 = rad(Rw0, Rb0, 1)
    A2, preA2 = rad(Rw0, Rb0, 2)
    o0 = _segsum(A0 * h0s0, dst, N) + h00 @ S0[0]
    o1 = _segsum((A1 * h0s0)[:, :, None] * Y1[:, None, :], dst, N)
    o2 = _segsum((A2 * h0s0)[:, :, None] * Y2[:, None, :], dst, N)

    def layer_fwd(h0, h1, h2, Rw, Rb, S):
        h0s = _gather(h0, src)
        h1s = _gather(h1, src)
        h2s = _gather(h2, src)
        R = [rad(Rw, Rb, p)[0] for p in range(5)]
        inv1 = jnp.sum(h1s * Y1[:, None, :], axis=-1)
        inv2 = jnp.sum(h2s * Y2[:, None, :], axis=-1)
        m0 = R[0] * h0s + R[1] * inv1 + R[2] * inv2
        n0 = _segsum(m0, dst, N) + h0 @ S[0]
        n1 = _segsum((R[3] * h0s)[:, :, None] * Y1[:, None, :], dst, N) \
            + jnp.einsum('ncm,cd->ndm', h1, S[1])
        n2 = _segsum((R[4] * h0s)[:, :, None] * Y2[:, None, :], dst, N) \
            + jnp.einsum('ncm,cd->ndm', h2, S[2])
        return n0, n1, n2

    hA = (o0, o1, o2)
    hB = layer_fwd(*hA, Rw1, Rb1, S1)
    hC = layer_fwd(*hB, Rw2, Rb2, S2)

    h0s = _gather(hC[0], src)
    h1s = _gather(hC[1], src)
    h2s = _gather(hC[2], src)
    R3 = [rad(Rw3, Rb3, p) for p in range(3)]
    inv1 = jnp.sum(h1s * Y1[:, None, :], axis=-1)
    inv2 = jnp.sum(h2s * Y2[:, None, :], axis=-1)
    m0 = R3[0][0] * h0s + R3[1][0] * inv1 + R3[2][0] * inv2
    f0 = _segsum(m0, dst, N) + hC[0] @ S3[0]

    z = f0 @ M1w + M1b
    a = _silu(z)
    e = a @ M2w + M2b
    total = jnp.sum(e)

    g_a = jnp.broadcast_to(M2w[:, 0][None, :], a.shape)
    g_z = g_a * _dsilu(z)
    g_f0 = g_z @ M1w.T

    gB = jnp.zeros_like(B)
    gY1 = jnp.zeros_like(Y1)
    gY2 = jnp.zeros_like(Y2)

    gm0 = _gather(g_f0, dst)
    g_h0 = g_f0 @ S3[0].T
    g_h0s = gm0 * R3[0][0]
    g_inv1 = gm0 * R3[1][0]
    g_inv2 = gm0 * R3[2][0]
    g_h1s = g_inv1[:, :, None] * Y1[:, None, :]
    g_h2s = g_inv2[:, :, None] * Y2[:, None, :]
    gY1 = gY1 + jnp.sum(g_inv1[:, :, None] * h1s, axis=1)
    gY2 = gY2 + jnp.sum(g_inv2[:, :, None] * h2s, axis=1)
    for p, gR in ((0, gm0 * h0s), (1, gm0 * inv1), (2, gm0 * inv2)):
        gB = gB + (gR * _dsilu(R3[p][1])) @ Rw3[p].T
    g_h0 = g_h0 + _segsum(g_h0s, src, N)
    g_h1 = _segsum(g_h1s, src, N)
    g_h2 = _segsum(g_h2s, src, N)

    def layer_bwd(g_n0, g_n1, g_n2, h, Rw, Rb, S, gB, gY1, gY2):
        h0s = _gather(h[0], src)
        h1s = _gather(h[1], src)
        h2s = _gather(h[2], src)
        R = [rad(Rw, Rb, p) for p in range(5)]
        inv1 = jnp.sum(h1s * Y1[:, None, :], axis=-1)
        inv2 = jnp.sum(h2s * Y2[:, None, :], axis=-1)
        gm0 = _gather(g_n0, dst)
        gmsg1 = _gather(g_n1, dst)
        gmsg2 = _gather(g_n2, dst)
        q1 = jnp.sum(gmsg1 * Y1[:, None, :], axis=-1)
        q2 = jnp.sum(gmsg2 * Y2[:, None, :], axis=-1)
        g_h0s = gm0 * R[0][0] + q1 * R[3][0] + q2 * R[4][0]
        g_inv1 = gm0 * R[1][0]
        g_inv2 = gm0 * R[2][0]
        g_h1s = g_inv1[:, :, None] * Y1[:, None, :]
        g_h2s = g_inv2[:, :, None] * Y2[:, None, :]
        gY1 = gY1 + jnp.sum(g_inv1[:, :, None] * h1s, axis=1) \
            + jnp.sum(gmsg1 * (R[3][0] * h0s)[:, :, None], axis=1)
        gY2 = gY2 + jnp.sum(g_inv2[:, :, None] * h2s, axis=1) \
            + jnp.sum(gmsg2 * (R[4][0] * h0s)[:, :, None], axis=1)
        for p, gR in ((0, gm0 * h0s), (1, gm0 * inv1), (2, gm0 * inv2),
                      (3, q1 * h0s), (4, q2 * h0s)):
            gB = gB + (gR * _dsilu(R[p][1])) @ Rw[p].T
        g_h0p = _segsum(g_h0s, src, N) + g_n0 @ S[0].T
        g_h1p = _segsum(g_h1s, src, N) + jnp.einsum('ndm,cd->ncm', g_n1, S[1])
        g_h2p = _segsum(g_h2s, src, N) + jnp.einsum('ndm,cd->ncm', g_n2, S[2])
        return g_h0p, g_h1p, g_h2p, gB, gY1, gY2

    g_h0, g_h1, g_h2, gB, gY1, gY2 = layer_bwd(
        g_h0, g_h1, g_h2, hB, Rw2, Rb2, S2, gB, gY1, gY2)
    g_h0, g_h1, g_h2, gB, gY1, gY2 = layer_bwd(
        g_h0, g_h1, g_h2, hA, Rw1, Rb1, S1, gB, gY1, gY2)

    gm = _gather(g_h0, dst)
    gmsg1 = _gather(g_h1, dst)
    gmsg2 = _gather(g_h2, dst)
    q1 = jnp.sum(gmsg1 * Y1[:, None, :], axis=-1)
    q2 = jnp.sum(gmsg2 * Y2[:, None, :], axis=-1)
    gY1 = gY1 + jnp.sum(gmsg1 * (A1 * h0s0)[:, :, None], axis=1)
    gY2 = gY2 + jnp.sum(gmsg2 * (A2 * h0s0)[:, :, None], axis=1)
    for pre, Rw_p, gR in ((preA0, Rw0[0], gm * h0s0), (preA1, Rw0[1], q1 * h0s0),
                          (preA2, Rw0[2], q2 * h0s0)):
        gB = gB + (gR * _dsilu(pre)) @ Rw_p.T

    x, y, z_ = u[:, 0], u[:, 1], u[:, 2]
    gu = jnp.stack([
        gY1[:, 0] + gY2[:, 0] * y + gY2[:, 3] * z_ + 2.0 * x * gY2[:, 4],
        gY1[:, 1] + gY2[:, 0] * x + gY2[:, 1] * z_ - 2.0 * y * gY2[:, 4],
        gY1[:, 2] + gY2[:, 1] * y + 6.0 * z_ * gY2[:, 2] + gY2[:, 3] * x,
    ], axis=-1)
    centers = jnp.asarray(_CENTERS, dtype=jnp.float32)
    gr = jnp.sum(gB * (-4.0 * (r[:, None] - centers[None, :]) * B), axis=-1)
    gvec = gr[:, None] * u + (gu - u * jnp.sum(u * gu, axis=-1, keepdims=True)) / r[:, None]
    g_pos = _segsum(gvec, dst, N) - _segsum(gvec, src, N)
    forces = -g_pos
    return jnp.reshape(total, (1, 1)), forces


def kernel(pos, emb, Rw0, Rb0, Rw1, Rb1, Rw2, Rb2, Rw3, Rb3,
           S0, S1, S2, S3, M1w, M1b, M2w, M2b, node_feats, edge_index):
    return manual_fwd_bwd(pos, emb, Rw0, Rb0, Rw1, Rb1, Rw2, Rb2, Rw3, Rb3,
                          S0, S1, S2, S3, M1w, M1b, M2w, M2b, node_feats, edge_index)


# speedup vs baseline: 14.1410x; 1.1602x over previous
import functools

import jax
import jax.numpy as jnp
from jax import lax
from jax.experimental import pallas as pl
from jax.experimental.pallas import tpu as pltpu
from jax.experimental.pallas import tpu_sc as plsc

NRBF = 8
C = 32
_CENTERS = [4.0 * k / (NRBF - 1) for k in range(NRBF)]

@functools.lru_cache(maxsize=None)
def _mesh():
    return plsc.VectorSubcoreMesh(core_axis_name="c", subcore_axis_name="s")


_NW = 32
_CH = 128


@functools.lru_cache(maxsize=None)
def _make_sc_gather(V, D, M):
    assert M % _CH == 0 and (D * 4) % 64 == 0
    chunks = M // _CH
    cpw = -(-chunks // _NW)

    @functools.partial(
        pl.kernel, mesh=_mesh(),
        compiler_params=pltpu.CompilerParams(use_tc_tiling_on_sc=False),
        out_type=jax.ShapeDtypeStruct((M, D), jnp.float32),
        scratch_types=[
            pltpu.VMEM((_CH,), jnp.int32),
            pltpu.VMEM((_CH, D), jnp.float32),
            pltpu.SemaphoreType.DMA,
        ],
    )
    def gather_k(table_hbm, idx_hbm, out_hbm, idx_v, rows_v, sem):
        w = lax.axis_index("s") * 2 + lax.axis_index("c")

        def body(i, carry):
            cid = w * cpw + i

            @pl.when(cid < chunks)
            def _():
                base = cid * _CH
                pltpu.sync_copy(idx_hbm.at[pl.ds(base, _CH)], idx_v)
                pltpu.async_copy(table_hbm.at[idx_v], rows_v, sem).wait()
                pltpu.sync_copy(rows_v, out_hbm.at[pl.ds(base, _CH)])
            return carry

        lax.fori_loop(0, cpw, body, 0)

    return gather_k


@functools.lru_cache(maxsize=None)
def _make_sc_segsum(P, M, N):
    assert M % (2 * _CH) == 0 and N % 16 == 0
    chunks = M // _CH
    per_core = chunks // 2
    cpt = -(-per_core // 16)
    NR = N // 16
    full, rem = NR // _CH, NR % _CH

    @functools.partial(
        pl.kernel, mesh=_mesh(),
        compiler_params=pltpu.CompilerParams(use_tc_tiling_on_sc=False),
        out_type=jax.ShapeDtypeStruct((2, P, N, 32), jnp.float32),
        scratch_types=[
            pltpu.VMEM((1, _CH), jnp.int32),
            pltpu.VMEM((_CH, 32), jnp.float32),
            pltpu.VMEM((_CH, 32), jnp.float32),
            pltpu.VMEM_SHARED((N, 32), jnp.float32),
            pltpu.SemaphoreType.DMA,
        ],
    )
    def segsum_k(data_hbm, idx_hbm, out_hbm, idxb, datb, zb, acc, sem):
        c = lax.axis_index("c")
        s = lax.axis_index("s")

        def zrow(i, carry):
            zb[i // 2, pl.ds((i % 2) * 16, 16)] = jnp.zeros((16,), jnp.float32)
            return carry

        lax.fori_loop(0, 2 * _CH, zrow, 0)

        row0 = s * NR

        for plane in range(P):
            def zloop(i, carry):
                pltpu.sync_copy(zb, acc.at[pl.ds(row0 + i * _CH, _CH)])
                return carry

            lax.fori_loop(0, full, zloop, 0)
            if rem:
                pltpu.sync_copy(zb.at[pl.ds(0, rem)],
                                acc.at[pl.ds(row0 + full * _CH, rem)])
            plsc.subcore_barrier()

            def sloop(i, carry):
                cid_local = s * cpt + i

                @pl.when(cid_local < per_core)
                def _():
                    cid = c * per_core + cid_local
                    base = cid * _CH
                    pltpu.sync_copy(idx_hbm.at[pl.ds(cid, 1)], idxb)
                    pltpu.sync_copy(data_hbm.at[plane, pl.ds(base, _CH)], datb)
                    pltpu.sync_copy(datb, acc.at[idxb.at[0]], add=True)
                return carry

            lax.fori_loop(0, cpt, sloop, 0)
            plsc.subcore_barrier()

            def dloop(i, carry):
                pltpu.sync_copy(acc.at[pl.ds(row0 + i * _CH, _CH)],
                                out_hbm.at[c, plane, pl.ds(row0 + i * _CH, _CH)])
                return carry

            lax.fori_loop(0, full, dloop, 0)
            if rem:
                pltpu.sync_copy(acc.at[pl.ds(row0 + full * _CH, rem)],
                                out_hbm.at[c, plane, pl.ds(row0 + full * _CH, rem)])
            plsc.subcore_barrier()

    return segsum_k


def _pad_rows(a, mult):
    n = a.shape[0]
    pad = (-n) % mult
    if pad:
        a = jnp.concatenate([a, jnp.zeros((pad,) + a.shape[1:], a.dtype)], axis=0)
    return a, n


def _gather(table, idx):
    shp = table.shape
    t2 = table.reshape(shp[0], -1)
    D = t2.shape[1]
    if D % 16 != 0:
        t2 = jnp.concatenate(
            [t2, jnp.zeros((shp[0], (-D) % 16), t2.dtype)], axis=1)
    idx_p, m = _pad_rows(idx, _CH)
    out = _make_sc_gather(shp[0], t2.shape[1], idx_p.shape[0])(t2, idx_p)
    return out[:m, :D].reshape((m,) + shp[1:])


def _segsum(data, idx, n):
    shp = data.shape
    if data.ndim == 2:
        planes = data[None]
    else:
        planes = jnp.moveaxis(data, -1, 0)
    P, W = planes.shape[0], planes.shape[-1]
    if W != 32:
        planes = jnp.concatenate(
            [planes, jnp.zeros(planes.shape[:-1] + ((-W) % 32,), planes.dtype)],
            axis=-1)
    out = _make_sc_segsum(P, shp[0], n)(planes, _seg_idx2d(idx))
    tot = (out[0] + out[1])[..., :W]
    if data.ndim == 2:
        return tot[0]
    return jnp.moveaxis(tot, 0, -1)


def _seg_idx2d(idx):
    chunks = idx.shape[0] // _CH
    per_core = chunks // 2
    cpt = -(-per_core // 16)
    idx2 = idx.reshape(chunks, _CH)
    pad = 2 * 16 * cpt - chunks
    if pad:
        idx2 = jnp.concatenate([idx2, jnp.zeros((pad, _CH), idx.dtype)], axis=0)
    return idx2


def _silu(x):
    s = jax.nn.sigmoid(x)
    return x * s


def _dsilu(x):
    s = jax.nn.sigmoid(x)
    return s * (1.0 + x * (1.0 - s))


def _segsum_planes(planes, idx, n):
    return _make_sc_segsum(planes.shape[0], planes.shape[1], n)(
        planes, _seg_idx2d(idx))


_INTERPRET = False


def _centers_row():
    i = lax.broadcasted_iota(jnp.int32, (1, NRBF), 1)
    return i.astype(jnp.float32) * (4.0 / (NRBF - 1))


def _geom_from_pos(ps, pd):
    vec = pd[:, :3] - ps[:, :3]
    r2 = jnp.sum(vec * vec, axis=1, keepdims=True) + 1e-9
    r = jnp.sqrt(r2)
    u = vec / r
    Bm = jnp.exp(-2.0 * (r - _centers_row()) ** 2)
    return vec, r, u, Bm


@functools.lru_cache(maxsize=None)
def _tc_geom_fwd(E, Eb):
    def body(ps_ref, pd_ref, out_ref):
        _, _, u, Bm = _geom_from_pos(ps_ref[...], pd_ref[...])
        x, y, z = u[:, 0:1], u[:, 1:2], u[:, 2:3]
        Y2 = jnp.concatenate(
            [x * y, y * z, 3.0 * z * z - 1.0, x * z, x * x - y * y], axis=1)
        out_ref[...] = jnp.concatenate([Bm, u, Y2], axis=1)

    return pl.pallas_call(
        body,
        grid=(E // Eb,),
        in_specs=[pl.BlockSpec((Eb, 16), lambda i: (i, 0)),
                  pl.BlockSpec((Eb, 16), lambda i: (i, 0))],
        out_specs=pl.BlockSpec((Eb, 16), lambda i: (i, 0)),
        out_shape=jax.ShapeDtypeStruct((E, 16), jnp.float32),
        interpret=_INTERPRET)


def _radials(B, Rw_ref, Rb_ref, nP):
    out = []
    for p in range(nP):
        pre = jnp.dot(B, Rw_ref[p], preferred_element_type=jnp.float32, precision=lax.Precision.HIGHEST) + Rb_ref[p]
        out.append((_silu(pre), pre))
    return out


def _plane(a, j):
    return a[:, 32 * j:32 * (j + 1)]


@functools.lru_cache(maxsize=None)
def _tc_edge_fwd(E, Eb, kind):
    nP = {"l0": 3, "mid": 5, "last": 3}[kind]
    Whs = 32 if kind == "l0" else 288
    Pout = 1 if kind == "last" else 9

    def body(geom_ref, hs_ref, Rw_ref, Rb_ref, msg_ref):
        g = geom_ref[...]
        B, Y1, Y2 = g[:, :8], g[:, 8:11], g[:, 11:16]
        hs = hs_ref[...]
        h0s = hs[:, :32]
        R = _radials(B, Rw_ref, Rb_ref, nP)
        if kind == "l0":
            msg_ref[0] = R[0][0] * h0s
            t1, t2 = R[1][0] * h0s, R[2][0] * h0s
            for m in range(3):
                msg_ref[1 + m] = t1 * Y1[:, m:m + 1]
            for m in range(5):
                msg_ref[4 + m] = t2 * Y2[:, m:m + 1]
        else:
            inv1 = sum(_plane(hs, 1 + m) * Y1[:, m:m + 1] for m in range(3))
            inv2 = sum(_plane(hs, 4 + m) * Y2[:, m:m + 1] for m in range(5))
            msg_ref[0] = R[0][0] * h0s + R[1][0] * inv1 + R[2][0] * inv2
            if kind == "mid":
                t3, t4 = R[3][0] * h0s, R[4][0] * h0s
                for m in range(3):
                    msg_ref[1 + m] = t3 * Y1[:, m:m + 1]
                for m in range(5):
                    msg_ref[4 + m] = t4 * Y2[:, m:m + 1]

    return pl.pallas_call(
        body,
        grid=(E // Eb,),
        in_specs=[pl.BlockSpec((Eb, 16), lambda i: (i, 0)),
                  pl.BlockSpec((Eb, Whs), lambda i: (i, 0)),
                  pl.BlockSpec((nP, 8, 32), lambda i: (0, 0, 0)),
                  pl.BlockSpec((nP, 32), lambda i: (0, 0))],
        out_specs=pl.BlockSpec((Pout, Eb, 32), lambda i: (0, i, 0)),
        out_shape=jax.ShapeDtypeStruct((Pout, E, 32), jnp.float32),
        interpret=_INTERPRET)


@functools.lru_cache(maxsize=None)
def _tc_node_combine(N, Nb, n_dense, Pout):
    def body(parts_ref, dh_ref, W_ref, out_ref):
        dh = dh_ref[...]
        for j in range(Pout):
            t = parts_ref[0, j] + parts_ref[1, j]
            if j < n_dense:
                t = t + jnp.dot(_plane(dh, j), W_ref[j],
                                preferred_element_type=jnp.float32, precision=lax.Precision.HIGHEST)
            out_ref[:, 32 * j:32 * (j + 1)] = t

    return pl.pallas_call(
        body,
        grid=(N // Nb,),
        in_specs=[pl.BlockSpec((2, Pout, Nb, 32), lambda i: (0, 0, i, 0)),
                  pl.BlockSpec((Nb, 32 * n_dense), lambda i: (i, 0)),
                  pl.BlockSpec((n_dense, 32, 32), lambda i: (0, 0, 0))],
        out_specs=pl.BlockSpec((Nb, 32 * Pout), lambda i: (i, 0)),
        out_shape=jax.ShapeDtypeStruct((N, 32 * Pout), jnp.float32),
        interpret=_INTERPRET)


@functools.lru_cache(maxsize=None)
def _tc_readout(N, Nb):
    def body(f0_ref, M1w_ref, M1b_ref, M2wr_ref, M1wT_ref, tot_ref, g_ref):
        f0 = f0_ref[...]
        z = jnp.dot(f0, M1w_ref[...], preferred_element_type=jnp.float32, precision=lax.Precision.HIGHEST) \
            + M1b_ref[...]
        s = jax.nn.sigmoid(z)
        a = z * s
        m2 = M2wr_ref[...]
        blk = jnp.sum(a * m2)

        @pl.when(pl.program_id(0) == 0)
        def _():
            tot_ref[...] = jnp.zeros((1, 1), jnp.float32)

        tot_ref[...] = tot_ref[...] + jnp.reshape(blk, (1, 1))
        g_z = m2 * (s * (1.0 + z * (1.0 - s)))
        g_ref[...] = jnp.dot(g_z, M1wT_ref[...],
                             preferred_element_type=jnp.float32, precision=lax.Precision.HIGHEST)

    return pl.pallas_call(
        body,
        grid=(N // Nb,),
        in_specs=[pl.BlockSpec((Nb, 32), lambda i: (i, 0)),
                  pl.BlockSpec((32, 32), lambda i: (0, 0)),
                  pl.BlockSpec((1, 32), lambda i: (0, 0)),
                  pl.BlockSpec((1, 32), lambda i: (0, 0)),
                  pl.BlockSpec((32, 32), lambda i: (0, 0))],
        out_specs=[pl.BlockSpec((1, 1), lambda i: (0, 0)),
                   pl.BlockSpec((Nb, 32), lambda i: (i, 0))],
        out_shape=[jax.ShapeDtypeStruct((1, 1), jnp.float32),
                   jax.ShapeDtypeStruct((N, 32), jnp.float32)],
        interpret=_INTERPRET)


@functools.lru_cache(maxsize=None)
def _tc_edge_bwd(E, Eb, kind):
    nP = {"l0": 3, "mid": 5, "last": 3}[kind]
    Whs = 32 if kind == "l0" else 288
    Wgm = 32 if kind == "last" else 288

    def body(geom_ref, hs_ref, gms_ref, ggin_ref, Rw_ref, Rb_ref, RwT_ref,
             *out_refs):
        g = geom_ref[...]
        B, Y1, Y2 = g[:, :8], g[:, 8:11], g[:, 11:16]
        hs = hs_ref[...]
        h0s = hs[:, :32]
        gms = gms_ref[...]
        R = _radials(B, Rw_ref, Rb_ref, nP)
        gm0 = gms[:, :32]
        gY1 = [None] * 3
        gY2 = [None] * 5
        if kind != "last":
            q1 = sum(_plane(gms, 1 + m) * Y1[:, m:m + 1] for m in range(3))
            q2 = sum(_plane(gms, 4 + m) * Y2[:, m:m + 1] for m in range(5))
        if kind == "l0":
            t1, t2 = R[1][0] * h0s, R[2][0] * h0s
            for m in range(3):
                gY1[m] = jnp.sum(_plane(gms, 1 + m) * t1, axis=1, keepdims=True)
            for m in range(5):
                gY2[m] = jnp.sum(_plane(gms, 4 + m) * t2, axis=1, keepdims=True)
            gR = [gm0 * h0s, q1 * h0s, q2 * h0s]
        else:
            gsc_ref = out_refs[0]
            inv1 = sum(_plane(hs, 1 + m) * Y1[:, m:m + 1] for m in range(3))
            inv2 = sum(_plane(hs, 4 + m) * Y2[:, m:m + 1] for m in range(5))
            g_inv1 = gm0 * R[1][0]
            g_inv2 = gm0 * R[2][0]
            if kind == "mid":
                gsc_ref[0] = gm0 * R[0][0] + q1 * R[3][0] + q2 * R[4][0]
                t3, t4 = R[3][0] * h0s, R[4][0] * h0s
                for m in range(3):
                    gY1[m] = jnp.sum(g_inv1 * _plane(hs, 1 + m)
                                     + _plane(gms, 1 + m) * t3,
                                     axis=1, keepdims=True)
                for m in range(5):
                    gY2[m] = jnp.sum(g_inv2 * _plane(hs, 4 + m)
                                     + _plane(gms, 4 + m) * t4,
                                     axis=1, keepdims=True)
                gR = [gm0 * h0s, gm0 * inv1, gm0 * inv2, q1 * h0s, q2 * h0s]
            else:
                gsc_ref[0] = gm0 * R[0][0]
                for m in range(3):
                    gY1[m] = jnp.sum(g_inv1 * _plane(hs, 1 + m),
                                     axis=1, keepdims=True)
                for m in range(5):
                    gY2[m] = jnp.sum(g_inv2 * _plane(hs, 4 + m),
                                     axis=1, keepdims=True)
                gR = [gm0 * h0s, gm0 * inv1, gm0 * inv2]
            for m in range(3):
                gsc_ref[1 + m] = g_inv1 * Y1[:, m:m + 1]
            for m in range(5):
                gsc_ref[4 + m] = g_inv2 * Y2[:, m:m + 1]
        gB = sum(jnp.dot(gR[p] * _dsilu(R[p][1]), RwT_ref[p],
                         preferred_element_type=jnp.float32, precision=lax.Precision.HIGHEST)
                 for p in range(nP))
        out_refs[-1][...] = ggin_ref[...] + jnp.concatenate(
            [gB] + gY1 + gY2, axis=1)

    if kind == "l0":
        out_specs = pl.BlockSpec((Eb, 16), lambda i: (i, 0))
        out_shape = jax.ShapeDtypeStruct((E, 16), jnp.float32)
    else:
        out_specs = [pl.BlockSpec((9, Eb, 32), lambda i: (0, i, 0)),
                     pl.BlockSpec((Eb, 16), lambda i: (i, 0))]
        out_shape = [jax.ShapeDtypeStruct((9, E, 32), jnp.float32),
                     jax.ShapeDtypeStruct((E, 16), jnp.float32)]

    return pl.pallas_call(
        body,
        grid=(E // Eb,),
        in_specs=[pl.BlockSpec((Eb, 16), lambda i: (i, 0)),
                  pl.BlockSpec((Eb, Whs), lambda i: (i, 0)),
                  pl.BlockSpec((Eb, Wgm), lambda i: (i, 0)),
                  pl.BlockSpec((Eb, 16), lambda i: (i, 0)),
                  pl.BlockSpec((nP, 8, 32), lambda i: (0, 0, 0)),
                  pl.BlockSpec((nP, 32), lambda i: (0, 0)),
                  pl.BlockSpec((nP, 32, 8), lambda i: (0, 0, 0))],
        out_specs=out_specs,
        out_shape=out_shape,
        interpret=_INTERPRET)


@functools.lru_cache(maxsize=None)
def _tc_geom_bwd(E, Eb):
    def body(ps_ref, pd_ref, gg_ref, out_ref):
        _, r, u, Bm = _geom_from_pos(ps_ref[...], pd_ref[...])
        gg = gg_ref[...]
        gB, gY1, gY2 = gg[:, :8], gg[:, 8:11], gg[:, 11:16]
        x, y, z = u[:, 0:1], u[:, 1:2], u[:, 2:3]
        gux = gY1[:, 0:1] + gY2[:, 0:1] * y + gY2[:, 3:4] * z \
            + 2.0 * x * gY2[:, 4:5]
        guy = gY1[:, 1:2] + gY2[:, 0:1] * x + gY2[:, 1:2] * z \
            - 2.0 * y * gY2[:, 4:5]
        guz = gY1[:, 2:3] + gY2[:, 1:2] * y + 6.0 * z * gY2[:, 2:3] \
            + gY2[:, 3:4] * x
        gu = jnp.concatenate([gux, guy, guz], axis=1)
        gr = jnp.sum(gB * (-4.0 * (r - _centers_row()) * Bm),
                     axis=1, keepdims=True)
        gvec = gr * u + (gu - u * jnp.sum(u * gu, axis=1, keepdims=True)) / r
        zpad = jnp.zeros((gvec.shape[0], 29), jnp.float32)
        out_ref[0] = jnp.concatenate([-gvec, zpad], axis=1)
        out_ref[1] = jnp.concatenate([gvec, zpad], axis=1)

    return pl.pallas_call(
        body,
        grid=(E // Eb,),
        in_specs=[pl.BlockSpec((Eb, 16), lambda i: (i, 0)),
                  pl.BlockSpec((Eb, 16), lambda i: (i, 0)),
                  pl.BlockSpec((Eb, 16), lambda i: (i, 0))],
        out_specs=pl.BlockSpec((2, Eb, 32), lambda i: (0, i, 0)),
        out_shape=jax.ShapeDtypeStruct((2, E, 32), jnp.float32),
        interpret=_INTERPRET)


@functools.lru_cache(maxsize=None)
def _tc_forces(N, Nb):
    def body(parts_ref, out_ref):
        out_ref[...] = (parts_ref[0, 0] + parts_ref[1, 0])[:, :3]

    return pl.pallas_call(
        body,
        grid=(N // Nb,),
        in_specs=[pl.BlockSpec((2, 1, Nb, 32), lambda i: (0, 0, i, 0))],
        out_specs=pl.BlockSpec((Nb, 3), lambda i: (i, 0)),
        out_shape=jax.ShapeDtypeStruct((N, 3), jnp.float32),
        interpret=_INTERPRET)


def kernel(pos, emb, Rw0, Rb0, Rw1, Rb1, Rw2, Rb2, Rw3, Rb3,
           S0, S1, S2, S3, M1w, M1b, M2w, M2b, node_feats, edge_index):
    N = pos.shape[0]
    E = edge_index.shape[1]
    Eb = 640 if E % 640 == 0 else E
    Nb = 1000 if N % 1000 == 0 else N
    src = edge_index[0]
    dst = edge_index[1]

    pos16 = jnp.concatenate([pos, jnp.zeros((N, 13), jnp.float32)], axis=1)
    ps = _gather(pos16, src)
    pd = _gather(pos16, dst)
    geom = _tc_geom_fwd(E, Eb)(ps, pd)

    h00 = _gather(emb, node_feats)
    h0s0 = _gather(h00, src)

    W9_1 = jnp.stack([S1[0]] + [S1[1]] * 3 + [S1[2]] * 5)
    W9_2 = jnp.stack([S2[0]] + [S2[1]] * 3 + [S2[2]] * 5)
    W9T_1 = jnp.swapaxes(W9_1, 1, 2)
    W9T_2 = jnp.swapaxes(W9_2, 1, 2)

    msg = _tc_edge_fwd(E, Eb, "l0")(geom, h0s0, Rw0, Rb0)
    parts = _segsum_planes(msg, dst, N)
    hA = _tc_node_combine(N, Nb, 1, 9)(parts, h00, S0)

    hs1 = _gather(hA, src)
    msg = _tc_edge_fwd(E, Eb, "mid")(geom, hs1, Rw1, Rb1)
    parts = _segsum_planes(msg, dst, N)
    hB = _tc_node_combine(N, Nb, 9, 9)(parts, hA, W9_1)

    hs2 = _gather(hB, src)
    msg = _tc_edge_fwd(E, Eb, "mid")(geom, hs2, Rw2, Rb2)
    parts = _segsum_planes(msg, dst, N)
    hC = _tc_node_combine(N, Nb, 9, 9)(parts, hB, W9_2)

    hs3 = _gather(hC, src)
    msg = _tc_edge_fwd(E, Eb, "last")(geom, hs3, Rw3, Rb3)
    parts = _segsum_planes(msg, dst, N)
    f0 = _tc_node_combine(N, Nb, 1, 1)(parts, hC[:, :32], S3)

    tot, g_f0 = _tc_readout(N, Nb)(
        f0, M1w, M1b[None, :], M2w[:, 0][None, :], M1w.T)
    total = tot[0, 0] + N * M2b[0]

    ggeom = jnp.zeros((E, 16), jnp.float32)

    gms = _gather(g_f0, dst)
    gsc, ggeom = _tc_edge_bwd(E, Eb, "last")(
        geom, hs3, gms, ggeom, Rw3, Rb3, jnp.swapaxes(Rw3, 1, 2))
    parts = _segsum_planes(gsc, src, N)
    g_h = _tc_node_combine(N, Nb, 1, 9)(
        parts, g_f0, jnp.swapaxes(S3, 1, 2))

    for hsX, Rw, Rb, W9T in ((hs2, Rw2, Rb2, W9T_2), (hs1, Rw1, Rb1, W9T_1)):
        gms = _gather(g_h, dst)
        gsc, ggeom = _tc_edge_bwd(E, Eb, "mid")(
            geom, hsX, gms, ggeom, Rw, Rb, jnp.swapaxes(Rw, 1, 2))
        parts = _segsum_planes(gsc, src, N)
        g_h = _tc_node_combine(N, Nb, 9, 9)(parts, g_h, W9T)

    gms = _gather(g_h, dst)
    ggeom = _tc_edge_bwd(E, Eb, "l0")(
        geom, h0s0, gms, ggeom, Rw0, Rb0, jnp.swapaxes(Rw0, 1, 2))

    gvpm = _tc_geom_bwd(E, Eb)(ps, pd, ggeom)
    parts = _segsum_planes(gvpm.reshape(1, 2 * E, 32),
                           jnp.concatenate([dst, src]), N)
    forces = _tc_forces(N, Nb)(parts)
    return jnp.reshape(total, (1, 1)), forces


# speedup vs baseline: 16.9509x; 1.1987x over previous
import functools

import jax
import jax.numpy as jnp
from jax import lax
from jax.experimental import pallas as pl
from jax.experimental.pallas import tpu as pltpu
from jax.experimental.pallas import tpu_sc as plsc

NRBF = 8
C = 32
_CENTERS = [4.0 * k / (NRBF - 1) for k in range(NRBF)]

@functools.lru_cache(maxsize=None)
def _mesh():
    return plsc.VectorSubcoreMesh(core_axis_name="c", subcore_axis_name="s")


_NW = 32
_CH = 128


@functools.lru_cache(maxsize=None)
def _make_sc_gather(V, D, M):
    assert M % _CH == 0 and (D * 4) % 64 == 0
    chunks = M // _CH
    cpw = -(-chunks // _NW)

    @functools.partial(
        pl.kernel, mesh=_mesh(),
        compiler_params=pltpu.CompilerParams(use_tc_tiling_on_sc=False),
        out_type=jax.ShapeDtypeStruct((M, D), jnp.float32),
        scratch_types=[
            pltpu.VMEM((_CH,), jnp.int32),
            pltpu.VMEM((_CH,), jnp.int32),
            pltpu.VMEM((_CH, D), jnp.float32),
            pltpu.VMEM((_CH, D), jnp.float32),
            pltpu.SemaphoreType.DMA,
            pltpu.SemaphoreType.DMA,
        ],
    )
    def gather_k(table_hbm, idx_hbm, out_hbm, idx0, idx1, rows0, rows1,
                 sem0, sem1):
        w = lax.axis_index("s") * 2 + lax.axis_index("c")
        lo = w * cpw
        n_valid = jnp.minimum(chunks - lo, cpw)
        idxs = (idx0, idx1)
        rows = (rows0, rows1)
        sems = (sem0, sem1)

        @pl.when(n_valid > 0)
        def _():
            pltpu.sync_copy(idx_hbm.at[pl.ds(lo * _CH, _CH)], idx0)
            pltpu.async_copy(table_hbm.at[idx0], rows0, sem0)

        def body(k, carry):
            for slot in (0, 1):
                i = 2 * k + slot

                @pl.when(i < n_valid)
                def _(i=i, slot=slot):
                    nxt = 1 - slot

                    @pl.when(i + 1 < n_valid)
                    def _():
                        pltpu.sync_copy(
                            idx_hbm.at[pl.ds((lo + i + 1) * _CH, _CH)],
                            idxs[nxt])
                        pltpu.async_copy(table_hbm.at[idxs[nxt]], rows[nxt],
                                         sems[nxt])

                    pltpu.make_async_copy(table_hbm.at[idxs[slot]],
                                          rows[slot], sems[slot]).wait()
                    pltpu.sync_copy(rows[slot],
                                    out_hbm.at[pl.ds((lo + i) * _CH, _CH)])
            return carry

        lax.fori_loop(0, (cpw + 1) // 2, body, 0)

    return gather_k


@functools.lru_cache(maxsize=None)
def _make_sc_segsum(P, M, N):
    assert M % (2 * _CH) == 0 and N % 16 == 0
    chunks = M // _CH
    per_core = chunks // 2
    cpt = -(-per_core // 16)
    NR = N // 16
    full, rem = NR // _CH, NR % _CH

    @functools.partial(
        pl.kernel, mesh=_mesh(),
        compiler_params=pltpu.CompilerParams(use_tc_tiling_on_sc=False),
        out_type=jax.ShapeDtypeStruct((2, P, N, 32), jnp.float32),
        scratch_types=[
            pltpu.VMEM((1, _CH), jnp.int32),
            pltpu.VMEM((1, _CH), jnp.int32),
            pltpu.VMEM((_CH, 32), jnp.float32),
            pltpu.VMEM((_CH, 32), jnp.float32),
            pltpu.VMEM((_CH, 32), jnp.float32),
            pltpu.VMEM_SHARED((N, 32), jnp.float32),
            pltpu.SemaphoreType.DMA,
            pltpu.SemaphoreType.DMA,
        ],
    )
    def segsum_k(data_hbm, idx_hbm, out_hbm, idxb0, idxb1, datb0, datb1,
                 zb, acc, sem0, sem1):
        c = lax.axis_index("c")
        s = lax.axis_index("s")

        def zrow(i, carry):
            zb[i // 2, pl.ds((i % 2) * 16, 16)] = jnp.zeros((16,), jnp.float32)
            return carry

        lax.fori_loop(0, 2 * _CH, zrow, 0)

        row0 = s * NR

        for plane in range(P):
            def zloop(i, carry):
                pltpu.sync_copy(zb, acc.at[pl.ds(row0 + i * _CH, _CH)])
                return carry

            lax.fori_loop(0, full, zloop, 0)
            if rem:
                pltpu.sync_copy(zb.at[pl.ds(0, rem)],
                                acc.at[pl.ds(row0 + full * _CH, rem)])
            plsc.subcore_barrier()

            idxs = (idxb0, idxb1)
            dats = (datb0, datb1)
            sems = (sem0, sem1)
            lo = c * per_core + s * cpt
            n_valid = jnp.minimum(per_core - s * cpt, cpt)

            def start_load(i, slot):
                pltpu.async_copy(idx_hbm.at[pl.ds(lo + i, 1)], idxs[slot],
                                 sems[slot])
                pltpu.async_copy(data_hbm.at[plane, pl.ds((lo + i) * _CH, _CH)],
                                 dats[slot], sems[slot])

            def wait_load(i, slot):
                pltpu.make_async_copy(idx_hbm.at[pl.ds(lo + i, 1)], idxs[slot],
                                      sems[slot]).wait()
                pltpu.make_async_copy(
                    data_hbm.at[plane, pl.ds((lo + i) * _CH, _CH)],
                    dats[slot], sems[slot]).wait()

            @pl.when(n_valid > 0)
            def _():
                start_load(0, 0)

            def sloop(k, carry):
                for slot in (0, 1):
                    i = 2 * k + slot

                    @pl.when(i < n_valid)
                    def _(i=i, slot=slot):
                        @pl.when(i + 1 < n_valid)
                        def _():
                            start_load(i + 1, 1 - slot)

                        wait_load(i, slot)
                        pltpu.sync_copy(dats[slot], acc.at[idxs[slot].at[0]],
                                        add=True)
                return carry

            lax.fori_loop(0, (cpt + 1) // 2, sloop, 0)
            plsc.subcore_barrier()

            def dloop(i, carry):
                pltpu.sync_copy(acc.at[pl.ds(row0 + i * _CH, _CH)],
                                out_hbm.at[c, plane, pl.ds(row0 + i * _CH, _CH)])
                return carry

            lax.fori_loop(0, full, dloop, 0)
            if rem:
                pltpu.sync_copy(acc.at[pl.ds(row0 + full * _CH, rem)],
                                out_hbm.at[c, plane, pl.ds(row0 + full * _CH, rem)])
            plsc.subcore_barrier()

    return segsum_k


def _pad_rows(a, mult):
    n = a.shape[0]
    pad = (-n) % mult
    if pad:
        a = jnp.concatenate([a, jnp.zeros((pad,) + a.shape[1:], a.dtype)], axis=0)
    return a, n


def _gather(table, idx):
    shp = table.shape
    t2 = table.reshape(shp[0], -1)
    D = t2.shape[1]
    if D % 16 != 0:
        t2 = jnp.concatenate(
            [t2, jnp.zeros((shp[0], (-D) % 16), t2.dtype)], axis=1)
    idx_p, m = _pad_rows(idx, _CH)
    out = _make_sc_gather(shp[0], t2.shape[1], idx_p.shape[0])(t2, idx_p)
    return out[:m, :D].reshape((m,) + shp[1:])


def _segsum(data, idx, n):
    shp = data.shape
    if data.ndim == 2:
        planes = data[None]
    else:
        planes = jnp.moveaxis(data, -1, 0)
    P, W = planes.shape[0], planes.shape[-1]
    if W != 32:
        planes = jnp.concatenate(
            [planes, jnp.zeros(planes.shape[:-1] + ((-W) % 32,), planes.dtype)],
            axis=-1)
    out = _make_sc_segsum(P, shp[0], n)(planes, _seg_idx2d(idx))
    tot = (out[0] + out[1])[..., :W]
    if data.ndim == 2:
        return tot[0]
    return jnp.moveaxis(tot, 0, -1)


def _seg_idx2d(idx):
    chunks = idx.shape[0] // _CH
    per_core = chunks // 2
    cpt = -(-per_core // 16)
    idx2 = idx.reshape(chunks, _CH)
    pad = 2 * 16 * cpt - chunks
    if pad:
        idx2 = jnp.concatenate([idx2, jnp.zeros((pad, _CH), idx.dtype)], axis=0)
    return idx2


def _silu(x):
    s = jax.nn.sigmoid(x)
    return x * s


def _dsilu(x):
    s = jax.nn.sigmoid(x)
    return s * (1.0 + x * (1.0 - s))


def _segsum_planes(planes, idx, n):
    return _make_sc_segsum(planes.shape[0], planes.shape[1], n)(
        planes, _seg_idx2d(idx))


_INTERPRET = False


def _centers_row():
    i = lax.broadcasted_iota(jnp.int32, (1, NRBF), 1)
    return i.astype(jnp.float32) * (4.0 / (NRBF - 1))


def _geom_from_pos(ps, pd):
    vec = pd[:, :3] - ps[:, :3]
    r2 = jnp.sum(vec * vec, axis=1, keepdims=True) + 1e-9
    r = jnp.sqrt(r2)
    u = vec / r
    Bm = jnp.exp(-2.0 * (r - _centers_row()) ** 2)
    return vec, r, u, Bm


@functools.lru_cache(maxsize=None)
def _tc_geom_fwd(E, Eb):
    def body(ps_ref, pd_ref, out_ref):
        _, _, u, Bm = _geom_from_pos(ps_ref[...], pd_ref[...])
        x, y, z = u[:, 0:1], u[:, 1:2], u[:, 2:3]
        Y2 = jnp.concatenate(
            [x * y, y * z, 3.0 * z * z - 1.0, x * z, x * x - y * y], axis=1)
        out_ref[...] = jnp.concatenate([Bm, u, Y2], axis=1)

    return pl.pallas_call(
        body,
        grid=(E // Eb,),
        in_specs=[pl.BlockSpec((Eb, 16), lambda i: (i, 0)),
                  pl.BlockSpec((Eb, 16), lambda i: (i, 0))],
        out_specs=pl.BlockSpec((Eb, 16), lambda i: (i, 0)),
        out_shape=jax.ShapeDtypeStruct((E, 16), jnp.float32),
        interpret=_INTERPRET)


def _radials(B, Rw_ref, Rb_ref, nP):
    out = []
    for p in range(nP):
        pre = jnp.dot(B, Rw_ref[p], preferred_element_type=jnp.float32, precision=lax.Precision.HIGHEST) + Rb_ref[p]
        out.append((_silu(pre), pre))
    return out


def _plane(a, j):
    return a[:, 32 * j:32 * (j + 1)]


@functools.lru_cache(maxsize=None)
def _tc_edge_fwd(E, Eb, kind):
    nP = {"l0": 3, "mid": 5, "last": 3}[kind]
    Whs = 32 if kind == "l0" else 288
    Pout = 1 if kind == "last" else 9

    def body(geom_ref, hs_ref, Rw_ref, Rb_ref, msg_ref):
        g = geom_ref[...]
        B, Y1, Y2 = g[:, :8], g[:, 8:11], g[:, 11:16]
        hs = hs_ref[...]
        h0s = hs[:, :32]
        R = _radials(B, Rw_ref, Rb_ref, nP)
        if kind == "l0":
            msg_ref[0] = R[0][0] * h0s
            t1, t2 = R[1][0] * h0s, R[2][0] * h0s
            for m in range(3):
                msg_ref[1 + m] = t1 * Y1[:, m:m + 1]
            for m in range(5):
                msg_ref[4 + m] = t2 * Y2[:, m:m + 1]
        else:
            inv1 = sum(_plane(hs, 1 + m) * Y1[:, m:m + 1] for m in range(3))
            inv2 = sum(_plane(hs, 4 + m) * Y2[:, m:m + 1] for m in range(5))
            msg_ref[0] = R[0][0] * h0s + R[1][0] * inv1 + R[2][0] * inv2
            if kind == "mid":
                t3, t4 = R[3][0] * h0s, R[4][0] * h0s
                for m in range(3):
                    msg_ref[1 + m] = t3 * Y1[:, m:m + 1]
                for m in range(5):
                    msg_ref[4 + m] = t4 * Y2[:, m:m + 1]

    return pl.pallas_call(
        body,
        grid=(E // Eb,),
        in_specs=[pl.BlockSpec((Eb, 16), lambda i: (i, 0)),
                  pl.BlockSpec((Eb, Whs), lambda i: (i, 0)),
                  pl.BlockSpec((nP, 8, 32), lambda i: (0, 0, 0)),
                  pl.BlockSpec((nP, 32), lambda i: (0, 0))],
        out_specs=pl.BlockSpec((Pout, Eb, 32), lambda i: (0, i, 0)),
        out_shape=jax.ShapeDtypeStruct((Pout, E, 32), jnp.float32),
        interpret=_INTERPRET)


@functools.lru_cache(maxsize=None)
def _tc_node_combine(N, Nb, n_dense, Pout):
    def body(parts_ref, dh_ref, W_ref, out_ref):
        dh = dh_ref[...]
        for j in range(Pout):
            t = parts_ref[0, j] + parts_ref[1, j]
            if j < n_dense:
                t = t + jnp.dot(_plane(dh, j), W_ref[j],
                                preferred_element_type=jnp.float32, precision=lax.Precision.HIGHEST)
            out_ref[:, 32 * j:32 * (j + 1)] = t

    return pl.pallas_call(
        body,
        grid=(N // Nb,),
        in_specs=[pl.BlockSpec((2, Pout, Nb, 32), lambda i: (0, 0, i, 0)),
                  pl.BlockSpec((Nb, 32 * n_dense), lambda i: (i, 0)),
                  pl.BlockSpec((n_dense, 32, 32), lambda i: (0, 0, 0))],
        out_specs=pl.BlockSpec((Nb, 32 * Pout), lambda i: (i, 0)),
        out_shape=jax.ShapeDtypeStruct((N, 32 * Pout), jnp.float32),
        interpret=_INTERPRET)


@functools.lru_cache(maxsize=None)
def _tc_readout(N, Nb):
    def body(f0_ref, M1w_ref, M1b_ref, M2wr_ref, M1wT_ref, tot_ref, g_ref):
        f0 = f0_ref[...]
        z = jnp.dot(f0, M1w_ref[...], preferred_element_type=jnp.float32, precision=lax.Precision.HIGHEST) \
            + M1b_ref[...]
        s = jax.nn.sigmoid(z)
        a = z * s
        m2 = M2wr_ref[...]
        blk = jnp.sum(a * m2)

        @pl.when(pl.program_id(0) == 0)
        def _():
            tot_ref[...] = jnp.zeros((1, 1), jnp.float32)

        tot_ref[...] = tot_ref[...] + jnp.reshape(blk, (1, 1))
        g_z = m2 * (s * (1.0 + z * (1.0 - s)))
        g_ref[...] = jnp.dot(g_z, M1wT_ref[...],
                             preferred_element_type=jnp.float32, precision=lax.Precision.HIGHEST)

    return pl.pallas_call(
        body,
        grid=(N // Nb,),
        in_specs=[pl.BlockSpec((Nb, 32), lambda i: (i, 0)),
                  pl.BlockSpec((32, 32), lambda i: (0, 0)),
                  pl.BlockSpec((1, 32), lambda i: (0, 0)),
                  pl.BlockSpec((1, 32), lambda i: (0, 0)),
                  pl.BlockSpec((32, 32), lambda i: (0, 0))],
        out_specs=[pl.BlockSpec((1, 1), lambda i: (0, 0)),
                   pl.BlockSpec((Nb, 32), lambda i: (i, 0))],
        out_shape=[jax.ShapeDtypeStruct((1, 1), jnp.float32),
                   jax.ShapeDtypeStruct((N, 32), jnp.float32)],
        interpret=_INTERPRET)


@functools.lru_cache(maxsize=None)
def _tc_edge_bwd(E, Eb, kind):
    nP = {"l0": 3, "mid": 5, "last": 3}[kind]
    Whs = 32 if kind == "l0" else 288
    Wgm = 32 if kind == "last" else 288

    def body(geom_ref, hs_ref, gms_ref, ggin_ref, Rw_ref, Rb_ref, RwT_ref,
             *out_refs):
        g = geom_ref[...]
        B, Y1, Y2 = g[:, :8], g[:, 8:11], g[:, 11:16]
        hs = hs_ref[...]
        h0s = hs[:, :32]
        gms = gms_ref[...]
        R = _radials(B, Rw_ref, Rb_ref, nP)
        gm0 = gms[:, :32]
        gY1 = [None] * 3
        gY2 = [None] * 5
        if kind != "last":
            q1 = sum(_plane(gms, 1 + m) * Y1[:, m:m + 1] for m in range(3))
            q2 = sum(_plane(gms, 4 + m) * Y2[:, m:m + 1] for m in range(5))
        if kind == "l0":
            t1, t2 = R[1][0] * h0s, R[2][0] * h0s
            for m in range(3):
                gY1[m] = jnp.sum(_plane(gms, 1 + m) * t1, axis=1, keepdims=True)
            for m in range(5):
                gY2[m] = jnp.sum(_plane(gms, 4 + m) * t2, axis=1, keepdims=True)
            gR = [gm0 * h0s, q1 * h0s, q2 * h0s]
        else:
            gsc_ref = out_refs[0]
            inv1 = sum(_plane(hs, 1 + m) * Y1[:, m:m + 1] for m in range(3))
            inv2 = sum(_plane(hs, 4 + m) * Y2[:, m:m + 1] for m in range(5))
            g_inv1 = gm0 * R[1][0]
            g_inv2 = gm0 * R[2][0]
            if kind == "mid":
                gsc_ref[0] = gm0 * R[0][0] + q1 * R[3][0] + q2 * R[4][0]
                t3, t4 = R[3][0] * h0s, R[4][0] * h0s
                for m in range(3):
                    gY1[m] = jnp.sum(g_inv1 * _plane(hs, 1 + m)
                                     + _plane(gms, 1 + m) * t3,
                                     axis=1, keepdims=True)
                for m in range(5):
                    gY2[m] = jnp.sum(g_inv2 * _plane(hs, 4 + m)
                                     + _plane(gms, 4 + m) * t4,
                                     axis=1, keepdims=True)
                gR = [gm0 * h0s, gm0 * inv1, gm0 * inv2, q1 * h0s, q2 * h0s]
            else:
                gsc_ref[0] = gm0 * R[0][0]
                for m in range(3):
                    gY1[m] = jnp.sum(g_inv1 * _plane(hs, 1 + m),
                                     axis=1, keepdims=True)
                for m in range(5):
                    gY2[m] = jnp.sum(g_inv2 * _plane(hs, 4 + m),
                                     axis=1, keepdims=True)
                gR = [gm0 * h0s, gm0 * inv1, gm0 * inv2]
            for m in range(3):
                gsc_ref[1 + m] = g_inv1 * Y1[:, m:m + 1]
            for m in range(5):
                gsc_ref[4 + m] = g_inv2 * Y2[:, m:m + 1]
        gB = sum(jnp.dot(gR[p] * _dsilu(R[p][1]), RwT_ref[p],
                         preferred_element_type=jnp.float32, precision=lax.Precision.HIGHEST)
                 for p in range(nP))
        out_refs[-1][...] = ggin_ref[...] + jnp.concatenate(
            [gB] + gY1 + gY2, axis=1)

    if kind == "l0":
        out_specs = pl.BlockSpec((Eb, 16), lambda i: (i, 0))
        out_shape = jax.ShapeDtypeStruct((E, 16), jnp.float32)
    else:
        out_specs = [pl.BlockSpec((9, Eb, 32), lambda i: (0, i, 0)),
                     pl.BlockSpec((Eb, 16), lambda i: (i, 0))]
        out_shape = [jax.ShapeDtypeStruct((9, E, 32), jnp.float32),
                     jax.ShapeDtypeStruct((E, 16), jnp.float32)]

    return pl.pallas_call(
        body,
        grid=(E // Eb,),
        in_specs=[pl.BlockSpec((Eb, 16), lambda i: (i, 0)),
                  pl.BlockSpec((Eb, Whs), lambda i: (i, 0)),
                  pl.BlockSpec((Eb, Wgm), lambda i: (i, 0)),
                  pl.BlockSpec((Eb, 16), lambda i: (i, 0)),
                  pl.BlockSpec((nP, 8, 32), lambda i: (0, 0, 0)),
                  pl.BlockSpec((nP, 32), lambda i: (0, 0)),
                  pl.BlockSpec((nP, 32, 8), lambda i: (0, 0, 0))],
        out_specs=out_specs,
        out_shape=out_shape,
        interpret=_INTERPRET)


@functools.lru_cache(maxsize=None)
def _tc_geom_bwd(E, Eb):
    def body(ps_ref, pd_ref, gg_ref, out_ref):
        _, r, u, Bm = _geom_from_pos(ps_ref[...], pd_ref[...])
        gg = gg_ref[...]
        gB, gY1, gY2 = gg[:, :8], gg[:, 8:11], gg[:, 11:16]
        x, y, z = u[:, 0:1], u[:, 1:2], u[:, 2:3]
        gux = gY1[:, 0:1] + gY2[:, 0:1] * y + gY2[:, 3:4] * z \
            + 2.0 * x * gY2[:, 4:5]
        guy = gY1[:, 1:2] + gY2[:, 0:1] * x + gY2[:, 1:2] * z \
            - 2.0 * y * gY2[:, 4:5]
        guz = gY1[:, 2:3] + gY2[:, 1:2] * y + 6.0 * z * gY2[:, 2:3] \
            + gY2[:, 3:4] * x
        gu = jnp.concatenate([gux, guy, guz], axis=1)
        gr = jnp.sum(gB * (-4.0 * (r - _centers_row()) * Bm),
                     axis=1, keepdims=True)
        gvec = gr * u + (gu - u * jnp.sum(u * gu, axis=1, keepdims=True)) / r
        zpad = jnp.zeros((gvec.shape[0], 29), jnp.float32)
        out_ref[0] = jnp.concatenate([-gvec, zpad], axis=1)
        out_ref[1] = jnp.concatenate([gvec, zpad], axis=1)

    return pl.pallas_call(
        body,
        grid=(E // Eb,),
        in_specs=[pl.BlockSpec((Eb, 16), lambda i: (i, 0)),
                  pl.BlockSpec((Eb, 16), lambda i: (i, 0)),
                  pl.BlockSpec((Eb, 16), lambda i: (i, 0))],
        out_specs=pl.BlockSpec((2, Eb, 32), lambda i: (0, i, 0)),
        out_shape=jax.ShapeDtypeStruct((2, E, 32), jnp.float32),
        interpret=_INTERPRET)


@functools.lru_cache(maxsize=None)
def _tc_forces(N, Nb):
    def body(parts_ref, out_ref):
        out_ref[...] = (parts_ref[0, 0] + parts_ref[1, 0])[:, :3]

    return pl.pallas_call(
        body,
        grid=(N // Nb,),
        in_specs=[pl.BlockSpec((2, 1, Nb, 32), lambda i: (0, 0, i, 0))],
        out_specs=pl.BlockSpec((Nb, 3), lambda i: (i, 0)),
        out_shape=jax.ShapeDtypeStruct((N, 3), jnp.float32),
        interpret=_INTERPRET)


def kernel(pos, emb, Rw0, Rb0, Rw1, Rb1, Rw2, Rb2, Rw3, Rb3,
           S0, S1, S2, S3, M1w, M1b, M2w, M2b, node_feats, edge_index):
    N = pos.shape[0]
    E = edge_index.shape[1]
    Eb = 1600 if E % 1600 == 0 else E
    Nb = 1000 if N % 1000 == 0 else N
    src = edge_index[0]
    dst = edge_index[1]

    pos16 = jnp.concatenate([pos, jnp.zeros((N, 13), jnp.float32)], axis=1)
    ps = _gather(pos16, src)
    pd = _gather(pos16, dst)
    geom = _tc_geom_fwd(E, Eb)(ps, pd)

    h00 = _gather(emb, node_feats)
    h0s0 = _gather(h00, src)

    W9_1 = jnp.stack([S1[0]] + [S1[1]] * 3 + [S1[2]] * 5)
    W9_2 = jnp.stack([S2[0]] + [S2[1]] * 3 + [S2[2]] * 5)
    W9T_1 = jnp.swapaxes(W9_1, 1, 2)
    W9T_2 = jnp.swapaxes(W9_2, 1, 2)

    msg = _tc_edge_fwd(E, Eb, "l0")(geom, h0s0, Rw0, Rb0)
    parts = _segsum_planes(msg, dst, N)
    hA = _tc_node_combine(N, Nb, 1, 9)(parts, h00, S0)

    hs1 = _gather(hA, src)
    msg = _tc_edge_fwd(E, Eb, "mid")(geom, hs1, Rw1, Rb1)
    parts = _segsum_planes(msg, dst, N)
    hB = _tc_node_combine(N, Nb, 9, 9)(parts, hA, W9_1)

    hs2 = _gather(hB, src)
    msg = _tc_edge_fwd(E, Eb, "mid")(geom, hs2, Rw2, Rb2)
    parts = _segsum_planes(msg, dst, N)
    hC = _tc_node_combine(N, Nb, 9, 9)(parts, hB, W9_2)

    hs3 = _gather(hC, src)
    msg = _tc_edge_fwd(E, Eb, "last")(geom, hs3, Rw3, Rb3)
    parts = _segsum_planes(msg, dst, N)
    f0 = _tc_node_combine(N, Nb, 1, 1)(parts, hC[:, :32], S3)

    tot, g_f0 = _tc_readout(N, Nb)(
        f0, M1w, M1b[None, :], M2w[:, 0][None, :], M1w.T)
    total = tot[0, 0] + N * M2b[0]

    ggeom = jnp.zeros((E, 16), jnp.float32)

    gms = _gather(g_f0, dst)
    gsc, ggeom = _tc_edge_bwd(E, Eb, "last")(
        geom, hs3, gms, ggeom, Rw3, Rb3, jnp.swapaxes(Rw3, 1, 2))
    parts = _segsum_planes(gsc, src, N)
    g_h = _tc_node_combine(N, Nb, 1, 9)(
        parts, g_f0, jnp.swapaxes(S3, 1, 2))

    for hsX, Rw, Rb, W9T in ((hs2, Rw2, Rb2, W9T_2), (hs1, Rw1, Rb1, W9T_1)):
        gms = _gather(g_h, dst)
        gsc, ggeom = _tc_edge_bwd(E, Eb, "mid")(
            geom, hsX, gms, ggeom, Rw, Rb, jnp.swapaxes(Rw, 1, 2))
        parts = _segsum_planes(gsc, src, N)
        g_h = _tc_node_combine(N, Nb, 9, 9)(parts, g_h, W9T)

    gms = _gather(g_h, dst)
    ggeom = _tc_edge_bwd(E, Eb, "l0")(
        geom, h0s0, gms, ggeom, Rw0, Rb0, jnp.swapaxes(Rw0, 1, 2))

    gvpm = _tc_geom_bwd(E, Eb)(ps, pd, ggeom)
    parts = _segsum_planes(gvpm.reshape(1, 2 * E, 32),
                           jnp.concatenate([dst, src]), N)
    forces = _tc_forces(N, Nb)(parts)
    return jnp.reshape(total, (1, 1)), forces
